# baseline (speedup 1.0000x reference)
import jax
import jax.numpy as jnp
from jax.experimental import pallas as pl
from jax.experimental.pallas import tpu as pltpu

_D = 768
_K = 1024
_TILE_T = 1024


def _cbn_block(cb_ref, cbn_ref):
    cb = cb_ref[...]
    cbn_ref[...] = jnp.sum(cb * cb, axis=1, keepdims=True)


def _vq_block(x_ref, w_ref, b_ref, cb_ref, cbn_ref, out_ref):
    cb = cb_ref[...]
    x = x_ref[...]
    w = w_ref[...]
    y = jnp.dot(w, x, preferred_element_type=jnp.float32) + b_ref[...]
    s = jnp.dot(cb, y, preferred_element_type=jnp.float32)
    xn = jnp.sum(y * y, axis=0, keepdims=True)
    dist = (xn - 2.0 * s) + cbn_ref[...]
    out_ref[...] = jnp.argmin(dist, axis=0)[None, :].astype(jnp.int32)


def kernel(ssl_content, proj_w, proj_b, codebook):
    x = ssl_content[0]
    t_len = x.shape[1]
    b2 = proj_b[:, None]
    cbn = pl.pallas_call(
        _cbn_block,
        out_shape=jax.ShapeDtypeStruct((_K, 1), jnp.float32),
    )(codebook)
    return pl.pallas_call(
        _vq_block,
        grid=(t_len // _TILE_T,),
        in_specs=[
            pl.BlockSpec((_D, _TILE_T), lambda i: (0, i)),
            pl.BlockSpec((_D, _D), lambda i: (0, 0)),
            pl.BlockSpec((_D, 1), lambda i: (0, 0)),
            pl.BlockSpec((_K, _D), lambda i: (0, 0)),
            pl.BlockSpec((_K, 1), lambda i: (0, 0)),
        ],
        out_specs=pl.BlockSpec((1, _TILE_T), lambda i: (0, i)),
        out_shape=jax.ShapeDtypeStruct((1, t_len), jnp.int32),
    )(x, proj_w, b2, codebook, cbn)


# speedup vs baseline: 1.0305x; 1.0305x over previous
import jax
import jax.numpy as jnp
from jax.experimental import pallas as pl
from jax.experimental.pallas import tpu as pltpu

_D = 768
_K = 1024
_TILE_T = 1024


def _vq_block(x_ref, w_ref, b_ref, cb_ref, out_ref, cbn_ref):
    x = x_ref[...]
    w = w_ref[...]
    y = jnp.dot(w, x, preferred_element_type=jnp.float32) + b_ref[...]
    cb = cb_ref[...]

    @pl.when(pl.program_id(0) == 0)
    def _():
        cbn_ref[...] = jnp.sum(cb * cb, axis=1, keepdims=True)

    s = jnp.dot(cb, y, preferred_element_type=jnp.float32)
    xn = jnp.sum(y * y, axis=0, keepdims=True)
    dist = (xn - 2.0 * s) + cbn_ref[...]
    out_ref[...] = jnp.argmin(dist, axis=0)[None, :].astype(jnp.int32)


def kernel(ssl_content, proj_w, proj_b, codebook):
    x = ssl_content[0]
    t_len = x.shape[1]
    b2 = proj_b[:, None]
    return pl.pallas_call(
        _vq_block,
        grid=(t_len // _TILE_T,),
        in_specs=[
            pl.BlockSpec((_D, _TILE_T), lambda i: (0, i)),
            pl.BlockSpec((_D, _D), lambda i: (0, 0)),
            pl.BlockSpec((_D, 1), lambda i: (0, 0)),
            pl.BlockSpec((_K, _D), lambda i: (0, 0)),
        ],
        out_specs=pl.BlockSpec((1, _TILE_T), lambda i: (0, i)),
        out_shape=jax.ShapeDtypeStruct((1, t_len), jnp.int32),
        scratch_shapes=[pltpu.VMEM((_K, 1), jnp.float32)],
    )(x, proj_w, b2, codebook)


# speedup vs baseline: 1.1217x; 1.0886x over previous
import jax
import jax.numpy as jnp
from jax.experimental import pallas as pl
from jax.experimental.pallas import tpu as pltpu

_D = 768
_K = 1024
_TILE_T = 2048


def _vq_block(x_ref, w_ref, b_ref, cb_ref, out_ref, cbn_ref):
    cb = cb_ref[...]

    @pl.when(pl.program_id(0) == 0)
    def _():
        cbn_ref[...] = jnp.sum(cb * cb, axis=1, keepdims=True)

    x = x_ref[...]
    w = w_ref[...]
    y = jnp.dot(w, x, preferred_element_type=jnp.float32) + b_ref[...]
    s = jnp.dot(cb, y, preferred_element_type=jnp.float32)
    xn = jnp.sum(y * y, axis=0, keepdims=True)
    dist = (xn - 2.0 * s) + cbn_ref[...]
    out_ref[...] = jnp.argmin(dist, axis=0)[None, :].astype(jnp.int32)


def kernel(ssl_content, proj_w, proj_b, codebook):
    x = ssl_content[0]
    t_len = x.shape[1]
    b2 = proj_b[:, None]
    return pl.pallas_call(
        _vq_block,
        grid=(t_len // _TILE_T,),
        in_specs=[
            pl.BlockSpec((_D, _TILE_T), lambda i: (0, i)),
            pl.BlockSpec((_D, _D), lambda i: (0, 0)),
            pl.BlockSpec((_D, 1), lambda i: (0, 0)),
            pl.BlockSpec((_K, _D), lambda i: (0, 0)),
        ],
        out_specs=pl.BlockSpec((1, _TILE_T), lambda i: (0, i)),
        out_shape=jax.ShapeDtypeStruct((1, t_len), jnp.int32),
        scratch_shapes=[pltpu.VMEM((_K, 1), jnp.float32)],
    )(x, proj_w, b2, codebook)
